# baseline (speedup 1.0000x reference)
import functools

import jax
import jax.numpy as jnp
from jax import lax
from jax.experimental import pallas as pl
from jax.experimental.pallas import tpu as pltpu
from jax.experimental.pallas import tpu_sc as plsc

SIZE2 = 256
DENSE_COLS = 256
NNZ = 1048576

L = 16
NWORKERS = 32
PER_W = NNZ // NWORKERS
CHUNK = 8192
NCHUNK = PER_W // CHUNK

TC_BLK = 8


def _sc_accum_body(idx_hbm, vals_hbm, out_hbm,
                   acc_v, rows_v, cols_v, vals_v, sems):
    wid = lax.axis_index("s") * 2 + lax.axis_index("c")
    base = wid * PER_W

    def start(k):
        buf = k % 2
        off = base + k * CHUNK
        pltpu.async_copy(idx_hbm.at[0, pl.ds(off, CHUNK)], rows_v.at[buf],
                         sems.at[buf])
        pltpu.async_copy(idx_hbm.at[1, pl.ds(off, CHUNK)], cols_v.at[buf],
                         sems.at[buf])
        pltpu.async_copy(vals_hbm.at[pl.ds(off, CHUNK)], vals_v.at[buf],
                         sems.at[buf])

    def drain(k):
        buf = k % 2
        off = base + k * CHUNK
        pltpu.make_async_copy(idx_hbm.at[0, pl.ds(off, CHUNK)], rows_v.at[buf],
                              sems.at[buf]).wait()
        pltpu.make_async_copy(idx_hbm.at[1, pl.ds(off, CHUNK)], cols_v.at[buf],
                              sems.at[buf]).wait()
        pltpu.make_async_copy(vals_hbm.at[pl.ds(off, CHUNK)], vals_v.at[buf],
                              sems.at[buf]).wait()

    start(0)

    @plsc.parallel_loop(0, SIZE2, unroll=2)
    def _zero(r):
        for cb in range(DENSE_COLS // L):
            acc_v[r, pl.ds(cb * L, L)] = jnp.zeros((L,), jnp.float32)

    for k in range(NCHUNK):
        if k + 1 < NCHUNK:
            start(k + 1)
        drain(k)
        buf = k % 2

        @plsc.parallel_loop(0, CHUNK // L, unroll=4)
        def _scatter(i):
            r = rows_v[buf, pl.ds(i * L, L)]
            c = cols_v[buf, pl.ds(i * L, L)]
            v = vals_v[buf, pl.ds(i * L, L)]
            plsc.addupdate_scatter(acc_v, [r, c], v)

    pltpu.sync_copy(acc_v, out_hbm.at[wid])


def _sc_accum(idx, vals):
    mesh = plsc.VectorSubcoreMesh(core_axis_name="c", subcore_axis_name="s")
    kern = functools.partial(
        pl.kernel,
        mesh=mesh,
        compiler_params=pltpu.CompilerParams(needs_layout_passes=False),
        out_type=jax.ShapeDtypeStruct((NWORKERS, SIZE2, DENSE_COLS),
                                      jnp.float32),
        scratch_types=[
            pltpu.VMEM((SIZE2, DENSE_COLS), jnp.float32),
            pltpu.VMEM((2, CHUNK), jnp.int32),
            pltpu.VMEM((2, CHUNK), jnp.int32),
            pltpu.VMEM((2, CHUNK), jnp.float32),
            pltpu.SemaphoreType.DMA((2,)),
        ],
    )(_sc_accum_body)
    return kern(idx, vals)


def _tc_body(partials_ref, xs_ref, b_ref, out_ref, small_ref):
    @pl.when(pl.program_id(0) == 0)
    def _():
        a = jnp.sum(partials_ref[...], axis=0)
        small_ref[...] = (
            jnp.dot(a, xs_ref[...], preferred_element_type=jnp.float32,
                    precision=lax.Precision.HIGHEST)
            + b_ref[...]
        )
    out_ref[...] = jnp.broadcast_to(small_ref[...][None],
                                    (TC_BLK, SIZE2, DENSE_COLS))


def _tc_finish(partials, xs, bb):
    return pl.pallas_call(
        _tc_body,
        grid=(SIZE2 // TC_BLK,),
        in_specs=[
            pl.BlockSpec((NWORKERS, SIZE2, DENSE_COLS), lambda i: (0, 0, 0)),
            pl.BlockSpec((SIZE2, DENSE_COLS), lambda i: (0, 0)),
            pl.BlockSpec((1, DENSE_COLS), lambda i: (0, 0)),
        ],
        out_specs=pl.BlockSpec((TC_BLK, SIZE2, DENSE_COLS), lambda i: (i, 0, 0)),
        out_shape=jax.ShapeDtypeStruct((SIZE2, SIZE2, DENSE_COLS), jnp.float32),
        scratch_shapes=[pltpu.VMEM((SIZE2, DENSE_COLS), jnp.float32)],
    )(partials, xs, bb)


def kernel(x, indices, values, b):
    idx = indices.astype(jnp.int32)
    vals = values.astype(jnp.float32)
    partials = _sc_accum(idx, vals)
    xs = lax.slice(x, (0, 0), (SIZE2, DENSE_COLS))
    bb = b.reshape(1, DENSE_COLS)
    return _tc_finish(partials, xs, bb)


# speedup vs baseline: 1.0545x; 1.0545x over previous
import functools

import jax
import jax.numpy as jnp
from jax import lax
from jax.experimental import pallas as pl
from jax.experimental.pallas import tpu as pltpu
from jax.experimental.pallas import tpu_sc as plsc

SIZE2 = 256
DENSE_COLS = 256
NNZ = 1048576

L = 16
NWORKERS = 32
PER_W = NNZ // NWORKERS
CHUNK = 8192
NCHUNK = PER_W // CHUNK

TC_BLK = 16


def _sc_accum_body(idx_hbm, vals_hbm, out_hbm,
                   acc_v, rows_v, cols_v, vals_v, sems):
    wid = lax.axis_index("s") * 2 + lax.axis_index("c")
    base = wid * PER_W

    def start(k):
        buf = k % 2
        off = base + k * CHUNK
        pltpu.async_copy(idx_hbm.at[0, pl.ds(off, CHUNK)], rows_v.at[buf],
                         sems.at[buf])
        pltpu.async_copy(idx_hbm.at[1, pl.ds(off, CHUNK)], cols_v.at[buf],
                         sems.at[buf])
        pltpu.async_copy(vals_hbm.at[pl.ds(off, CHUNK)], vals_v.at[buf],
                         sems.at[buf])

    def drain(k):
        buf = k % 2
        off = base + k * CHUNK
        pltpu.make_async_copy(idx_hbm.at[0, pl.ds(off, CHUNK)], rows_v.at[buf],
                              sems.at[buf]).wait()
        pltpu.make_async_copy(idx_hbm.at[1, pl.ds(off, CHUNK)], cols_v.at[buf],
                              sems.at[buf]).wait()
        pltpu.make_async_copy(vals_hbm.at[pl.ds(off, CHUNK)], vals_v.at[buf],
                              sems.at[buf]).wait()

    start(0)

    @plsc.parallel_loop(0, SIZE2, unroll=2)
    def _zero(r):
        for cb in range(DENSE_COLS // L):
            acc_v[r, pl.ds(cb * L, L)] = jnp.zeros((L,), jnp.float32)

    for k in range(NCHUNK):
        if k + 1 < NCHUNK:
            start(k + 1)
        drain(k)
        buf = k % 2

        @plsc.parallel_loop(0, CHUNK // L, unroll=8)
        def _scatter(i):
            r = rows_v[buf, pl.ds(i * L, L)]
            c = cols_v[buf, pl.ds(i * L, L)]
            v = vals_v[buf, pl.ds(i * L, L)]
            plsc.addupdate_scatter(acc_v, [r, c], v)

    pltpu.sync_copy(acc_v, out_hbm.at[wid])


def _sc_accum(idx, vals):
    mesh = plsc.VectorSubcoreMesh(core_axis_name="c", subcore_axis_name="s")
    kern = functools.partial(
        pl.kernel,
        mesh=mesh,
        compiler_params=pltpu.CompilerParams(needs_layout_passes=False),
        out_type=jax.ShapeDtypeStruct((NWORKERS, SIZE2, DENSE_COLS),
                                      jnp.float32),
        scratch_types=[
            pltpu.VMEM((SIZE2, DENSE_COLS), jnp.float32),
            pltpu.VMEM((2, CHUNK), jnp.int32),
            pltpu.VMEM((2, CHUNK), jnp.int32),
            pltpu.VMEM((2, CHUNK), jnp.float32),
            pltpu.SemaphoreType.DMA((2,)),
        ],
    )(_sc_accum_body)
    return kern(idx, vals)


def _tc_body(partials_ref, xs_ref, b_ref, out_ref, small_ref):
    @pl.when(pl.program_id(0) == 0)
    def _():
        a = jnp.sum(partials_ref[...], axis=0)
        small_ref[...] = (
            jnp.dot(a, xs_ref[...], preferred_element_type=jnp.float32,
                    precision=lax.Precision.HIGHEST)
            + b_ref[...]
        )
    out_ref[...] = jnp.broadcast_to(small_ref[...][None],
                                    (TC_BLK, SIZE2, DENSE_COLS))


def _tc_finish(partials, xs, bb):
    return pl.pallas_call(
        _tc_body,
        grid=(SIZE2 // TC_BLK,),
        in_specs=[
            pl.BlockSpec((NWORKERS, SIZE2, DENSE_COLS), lambda i: (0, 0, 0)),
            pl.BlockSpec((SIZE2, DENSE_COLS), lambda i: (0, 0)),
            pl.BlockSpec((1, DENSE_COLS), lambda i: (0, 0)),
        ],
        out_specs=pl.BlockSpec((TC_BLK, SIZE2, DENSE_COLS), lambda i: (i, 0, 0)),
        out_shape=jax.ShapeDtypeStruct((SIZE2, SIZE2, DENSE_COLS), jnp.float32),
        scratch_shapes=[pltpu.VMEM((SIZE2, DENSE_COLS), jnp.float32)],
    )(partials, xs, bb)


def kernel(x, indices, values, b):
    idx = indices.astype(jnp.int32)
    vals = values.astype(jnp.float32)
    partials = _sc_accum(idx, vals)
    xs = lax.slice(x, (0, 0), (SIZE2, DENSE_COLS))
    bb = b.reshape(1, DENSE_COLS)
    return _tc_finish(partials, xs, bb)


# speedup vs baseline: 1.0748x; 1.0192x over previous
import functools

import jax
import jax.numpy as jnp
from jax import lax
from jax.experimental import pallas as pl
from jax.experimental.pallas import tpu as pltpu
from jax.experimental.pallas import tpu_sc as plsc

SIZE2 = 256
DENSE_COLS = 256
NNZ = 1048576

L = 16
NWORKERS = 32
PER_W = NNZ // NWORKERS
CHUNK = 8192
NCHUNK = PER_W // CHUNK

TC_BLK = 16


def _sc_accum_body(idx_hbm, vals_hbm, out_hbm,
                   acc_v, rc_v, vals_v, sems):
    wid = lax.axis_index("s") * 2 + lax.axis_index("c")
    base = wid * PER_W

    def start(k):
        buf = k % 2
        off = base + k * CHUNK
        pltpu.async_copy(idx_hbm.at[:, pl.ds(off, CHUNK)], rc_v.at[buf],
                         sems.at[buf])
        pltpu.async_copy(vals_hbm.at[pl.ds(off, CHUNK)], vals_v.at[buf],
                         sems.at[buf])

    def drain(k):
        buf = k % 2
        off = base + k * CHUNK
        pltpu.make_async_copy(idx_hbm.at[:, pl.ds(off, CHUNK)], rc_v.at[buf],
                              sems.at[buf]).wait()
        pltpu.make_async_copy(vals_hbm.at[pl.ds(off, CHUNK)], vals_v.at[buf],
                              sems.at[buf]).wait()

    start(0)

    @plsc.parallel_loop(0, SIZE2, unroll=2)
    def _zero(r):
        for cb in range(DENSE_COLS // L):
            acc_v[r, pl.ds(cb * L, L)] = jnp.zeros((L,), jnp.float32)

    for k in range(NCHUNK):
        if k + 1 < NCHUNK:
            start(k + 1)
        drain(k)
        buf = k % 2

        @plsc.parallel_loop(0, CHUNK // L, unroll=4)
        def _scatter(i):
            r = rc_v[buf, 0, pl.ds(i * L, L)]
            c = rc_v[buf, 1, pl.ds(i * L, L)]
            v = vals_v[buf, pl.ds(i * L, L)]
            plsc.addupdate_scatter(acc_v, [r, c], v)

    pltpu.sync_copy(acc_v, out_hbm.at[wid])


def _sc_accum(idx, vals):
    mesh = plsc.VectorSubcoreMesh(core_axis_name="c", subcore_axis_name="s")
    kern = functools.partial(
        pl.kernel,
        mesh=mesh,
        compiler_params=pltpu.CompilerParams(needs_layout_passes=False),
        out_type=jax.ShapeDtypeStruct((NWORKERS, SIZE2, DENSE_COLS),
                                      jnp.float32),
        scratch_types=[
            pltpu.VMEM((SIZE2, DENSE_COLS), jnp.float32),
            pltpu.VMEM((2, 2, CHUNK), jnp.int32),
            pltpu.VMEM((2, CHUNK), jnp.float32),
            pltpu.SemaphoreType.DMA((2,)),
        ],
    )(_sc_accum_body)
    return kern(idx, vals)


def _tc_body(partials_ref, xs_ref, b_ref, out_ref, small_ref):
    @pl.when(pl.program_id(0) == 0)
    def _():
        a = jnp.sum(partials_ref[...], axis=0)
        small_ref[...] = (
            jnp.dot(a, xs_ref[...], preferred_element_type=jnp.float32,
                    precision=lax.Precision.HIGHEST)
            + b_ref[...]
        )
    out_ref[...] = jnp.broadcast_to(small_ref[...][None],
                                    (TC_BLK, SIZE2, DENSE_COLS))


def _tc_finish(partials, xs, bb):
    return pl.pallas_call(
        _tc_body,
        grid=(SIZE2 // TC_BLK,),
        in_specs=[
            pl.BlockSpec((NWORKERS, SIZE2, DENSE_COLS), lambda i: (0, 0, 0)),
            pl.BlockSpec((SIZE2, DENSE_COLS), lambda i: (0, 0)),
            pl.BlockSpec((1, DENSE_COLS), lambda i: (0, 0)),
        ],
        out_specs=pl.BlockSpec((TC_BLK, SIZE2, DENSE_COLS), lambda i: (i, 0, 0)),
        out_shape=jax.ShapeDtypeStruct((SIZE2, SIZE2, DENSE_COLS), jnp.float32),
        scratch_shapes=[pltpu.VMEM((SIZE2, DENSE_COLS), jnp.float32)],
    )(partials, xs, bb)


def kernel(x, indices, values, b):
    idx = indices.astype(jnp.int32)
    vals = values.astype(jnp.float32)
    partials = _sc_accum(idx, vals)
    xs = lax.slice(x, (0, 0), (SIZE2, DENSE_COLS))
    bb = b.reshape(1, DENSE_COLS)
    return _tc_finish(partials, xs, bb)


# speedup vs baseline: 1.0860x; 1.0105x over previous
import functools

import jax
import jax.numpy as jnp
from jax import lax
from jax.experimental import pallas as pl
from jax.experimental.pallas import tpu as pltpu
from jax.experimental.pallas import tpu_sc as plsc

SIZE2 = 256
DENSE_COLS = 256
NNZ = 1048576

L = 16
NWORKERS = 32
PER_W = NNZ // NWORKERS
CHUNK = 4096
NCHUNK = PER_W // CHUNK

TC_BLK = 16


def _sc_accum_body(idx_hbm, vals_hbm, out_hbm,
                   acc_v, rc_v, vals_v, sems):
    wid = lax.axis_index("s") * 2 + lax.axis_index("c")
    base = wid * PER_W

    def start(k):
        buf = k % 2
        off = base + k * CHUNK
        pltpu.async_copy(idx_hbm.at[:, pl.ds(off, CHUNK)], rc_v.at[buf],
                         sems.at[buf])
        pltpu.async_copy(vals_hbm.at[pl.ds(off, CHUNK)], vals_v.at[buf],
                         sems.at[buf])

    def drain(k):
        buf = k % 2
        off = base + k * CHUNK
        pltpu.make_async_copy(idx_hbm.at[:, pl.ds(off, CHUNK)], rc_v.at[buf],
                              sems.at[buf]).wait()
        pltpu.make_async_copy(vals_hbm.at[pl.ds(off, CHUNK)], vals_v.at[buf],
                              sems.at[buf]).wait()

    start(0)

    @plsc.parallel_loop(0, SIZE2, unroll=2)
    def _zero(r):
        for cb in range(DENSE_COLS // L):
            acc_v[r, pl.ds(cb * L, L)] = jnp.zeros((L,), jnp.float32)

    for k in range(NCHUNK):
        if k + 1 < NCHUNK:
            start(k + 1)
        drain(k)
        buf = k % 2

        @plsc.parallel_loop(0, CHUNK // L, unroll=4)
        def _scatter(i):
            r = rc_v[buf, 0, pl.ds(i * L, L)]
            c = rc_v[buf, 1, pl.ds(i * L, L)]
            v = vals_v[buf, pl.ds(i * L, L)]
            plsc.addupdate_scatter(acc_v, [r, c], v)

    pltpu.sync_copy(acc_v, out_hbm.at[wid])


def _sc_accum(idx, vals):
    mesh = plsc.VectorSubcoreMesh(core_axis_name="c", subcore_axis_name="s")
    kern = functools.partial(
        pl.kernel,
        mesh=mesh,
        compiler_params=pltpu.CompilerParams(needs_layout_passes=False),
        out_type=jax.ShapeDtypeStruct((NWORKERS, SIZE2, DENSE_COLS),
                                      jnp.float32),
        scratch_types=[
            pltpu.VMEM((SIZE2, DENSE_COLS), jnp.float32),
            pltpu.VMEM((2, 2, CHUNK), jnp.int32),
            pltpu.VMEM((2, CHUNK), jnp.float32),
            pltpu.SemaphoreType.DMA((2,)),
        ],
    )(_sc_accum_body)
    return kern(idx, vals)


def _tc_body(partials_ref, xs_ref, b_ref, out_ref, small_ref):
    @pl.when(pl.program_id(0) == 0)
    def _():
        a = jnp.sum(partials_ref[...], axis=0)
        small_ref[...] = (
            jnp.dot(a, xs_ref[...], preferred_element_type=jnp.float32,
                    precision=lax.Precision.HIGHEST)
            + b_ref[...]
        )
    out_ref[...] = jnp.broadcast_to(small_ref[...][None],
                                    (TC_BLK, SIZE2, DENSE_COLS))


def _tc_finish(partials, xs, bb):
    return pl.pallas_call(
        _tc_body,
        grid=(SIZE2 // TC_BLK,),
        in_specs=[
            pl.BlockSpec((NWORKERS, SIZE2, DENSE_COLS), lambda i: (0, 0, 0)),
            pl.BlockSpec((SIZE2, DENSE_COLS), lambda i: (0, 0)),
            pl.BlockSpec((1, DENSE_COLS), lambda i: (0, 0)),
        ],
        out_specs=pl.BlockSpec((TC_BLK, SIZE2, DENSE_COLS), lambda i: (i, 0, 0)),
        out_shape=jax.ShapeDtypeStruct((SIZE2, SIZE2, DENSE_COLS), jnp.float32),
        scratch_shapes=[pltpu.VMEM((SIZE2, DENSE_COLS), jnp.float32)],
    )(partials, xs, bb)


def kernel(x, indices, values, b):
    idx = indices.astype(jnp.int32)
    vals = values.astype(jnp.float32)
    partials = _sc_accum(idx, vals)
    xs = lax.slice(x, (0, 0), (SIZE2, DENSE_COLS))
    bb = b.reshape(1, DENSE_COLS)
    return _tc_finish(partials, xs, bb)
